# baseline (device time: 46172 ns/iter reference)
import jax
import jax.numpy as jnp
from jax import lax
from jax.experimental import pallas as pl
from jax.experimental.pallas import tpu as pltpu

B = 32
H = 16
D = 128
BS = 32
NB = 256
P_LOCAL = 256
P_DEV = 128
KB_PAGES = 64
KB_TOK = KB_PAGES * BS
N_KB = P_DEV // KB_PAGES
G = 4
HG = H // G
GB = HG * B
GD = HG * D
NEG = -1e30
LOG2E = 1.4426950408889634
SCALE = D ** -0.5
GD_A = GD + B
MESH = pl.DeviceIdType.MESH

TOTAL_STEPS = G * N_KB


def _attn_body(xref, qbd_ref, k_hbm, v_hbm, bt_ref, out_ref,
               bias_ref, kbuf, vbuf, m_sc, l_sc,
               acc_cur, m_all, l_all, s_acc, ml_send,
               r_acc, r_ml, dsems,
               asend, arecv, mlsend, mlrecv):
    step = pl.program_id(0)
    kb = step % N_KB
    my_x = lax.axis_index("x")
    my_y = lax.axis_index("y")
    peers = (
        (my_x, 1 - my_y),
        (1 - my_x, my_y),
        (1 - my_x, 1 - my_y),
    )

    def dma_descs(s, slot):
        g = s // N_KB
        row0 = (xref[0] + s % N_KB) * KB_TOK
        descs = []
        for t, (hbm, buf) in enumerate(((k_hbm, kbuf), (v_hbm, vbuf))):
            for a in range(HG):
                descs.append(pltpu.make_async_copy(
                    hbm.at[pl.ds(row0, KB_TOK), g * HG + a],
                    buf.at[slot, :, pl.ds(a * D, D)],
                    dsems.at[slot, t * HG + a]))
        return descs

    def acc_rdma(g, j):
        return pltpu.make_async_remote_copy(
            src_ref=s_acc.at[pl.ds(g * HG, HG)],
            dst_ref=r_acc.at[j, pl.ds(g * HG, HG)],
            send_sem=asend.at[g, j], recv_sem=arecv.at[g, j],
            device_id=peers[j], device_id_type=MESH)

    def ml_rdma(j):
        return pltpu.make_async_remote_copy(
            src_ref=ml_send, dst_ref=r_ml.at[j],
            send_sem=mlsend.at[j], recv_sem=mlrecv.at[j],
            device_id=peers[j], device_id_type=MESH)

    @pl.when(step == 0)
    def _prologue():
        for d in dma_descs(0, 0):
            d.start()
        barrier = pltpu.get_barrier_semaphore()
        for nbr in peers:
            pl.semaphore_signal(barrier, inc=1, device_id=nbr,
                                device_id_type=MESH)
        pl.semaphore_wait(barrier, 3)

    @pl.when(step + 1 < TOTAL_STEPS)
    def _prefetch():
        for d in dma_descs(step + 1, (step + 1) % 2):
            d.start()

    @pl.when(step == 0)
    def _bias():
        bt = bt_ref[...]
        base = my_y * P_LOCAL + xref[0] * KB_PAGES
        CH = 32
        for c in range(P_DEV // CH):
            pages = base + c * CH + lax.broadcasted_iota(
                jnp.int32, (CH, 1, 1), 0)
            eq = bt[None, :, :] == pages
            cnt = jnp.sum(jnp.where(eq, 1.0, 0.0), axis=2)
            bias_ref[c * CH:(c + 1) * CH, :] = jnp.where(
                cnt > 0.5, jnp.log2(cnt), NEG)

    @pl.when(kb == 0)
    def _init():
        m_sc[...] = jnp.full((GB, 1), NEG, jnp.float32)
        l_sc[...] = jnp.zeros((GB, 1), jnp.float32)

    bias_blk = bias_ref[pl.ds(kb * KB_PAGES, KB_PAGES), :]
    rows = lax.broadcasted_iota(jnp.int32, (KB_PAGES, KB_TOK), 0)
    cols = lax.broadcasted_iota(jnp.int32, (KB_PAGES, KB_TOK), 1)
    expand = jnp.where(cols // BS == rows, 1.0, 0.0)
    bias_cols = lax.dot_general(
        expand, bias_blk, (((0,), (0,)), ((), ())),
        preferred_element_type=jnp.float32)

    slot = step % 2
    for d in dma_descs(step, slot):
        d.wait()
    kbuf[slot, :, GD:GD_A] = bias_cols

    g_dyn = step // N_KB
    kg = kbuf[slot].astype(jnp.bfloat16)
    vg = vbuf[slot].astype(jnp.bfloat16)
    s = lax.dot_general(
        qbd_ref[0], kg, (((1,), (1,)), ((), ())),
        preferred_element_type=jnp.float32)
    m_old = m_sc[...]
    m_new = jnp.maximum(m_old, jnp.max(s, axis=1, keepdims=True))
    p = jnp.exp2(s - m_new)
    corr = jnp.exp2(m_old - m_new)
    m_sc[...] = m_new
    l_sc[...] = l_sc[...] * corr + jnp.sum(p, axis=1, keepdims=True)
    o = lax.dot_general(
        p.astype(jnp.bfloat16), vg, (((1,), (0,)), ((), ())),
        preferred_element_type=jnp.float32)

    @pl.when(kb == 0)
    def _acc_first():
        for a in range(HG):
            acc_cur[a] = o[a * B:(a + 1) * B, a * D:(a + 1) * D]

    @pl.when(kb != 0)
    def _acc_rest():
        for a in range(HG):
            acc_cur[a] = (acc_cur[a] * corr[a * B:(a + 1) * B]
                          + o[a * B:(a + 1) * B, a * D:(a + 1) * D])

    @pl.when(kb == N_KB - 1)
    def _group_done():
        m_all[pl.ds(g_dyn * HG, HG)] = m_sc[...].reshape(HG, B, 1)
        l_all[pl.ds(g_dyn * HG, HG)] = l_sc[...].reshape(HG, B, 1)
        rows_g = pl.ds(g_dyn * HG, HG)
        s_acc[rows_g] = acc_cur[...].astype(jnp.bfloat16)
        for j in range(3):
            pltpu.make_async_remote_copy(
                src_ref=s_acc.at[rows_g],
                dst_ref=r_acc.at[j, rows_g],
                send_sem=asend.at[g_dyn, j], recv_sem=arecv.at[g_dyn, j],
                device_id=peers[j], device_id_type=MESH).start()

    @pl.when(step == TOTAL_STEPS - 1)
    def _finish():
        ml_send[0:H] = m_all[...]
        ml_send[H:2 * H] = l_all[...]
        for j in range(3):
            ml_rdma(j).start()
        for g in range(G):
            for j in range(3):
                acc_rdma(g, j).wait()
        for j in range(3):
            ml_rdma(j).wait()

        m = m_all[...]
        r_m = [r_ml[j, 0:H] for j in range(3)]
        r_l = [r_ml[j, H:2 * H] for j in range(3)]
        mt = jnp.maximum(jnp.maximum(m, r_m[0]),
                         jnp.maximum(r_m[1], r_m[2]))
        w = jnp.exp2(m - mt)
        lt = w * l_all[...]
        ot = w * s_acc[...].astype(jnp.float32)
        for j in range(3):
            w = jnp.exp2(r_m[j] - mt)
            lt = lt + w * r_l[j]
            ot = ot + w * r_acc[j].astype(jnp.float32)
        out_ref[...] = ot / lt


def _attention(xarr, qbd, k, v, btv):
    grid_spec = pltpu.PrefetchScalarGridSpec(
        num_scalar_prefetch=1,
        grid=(TOTAL_STEPS,),
        in_specs=[
            pl.BlockSpec((1, GB, GD_A), lambda i, xr: (i // N_KB, 0, 0)),
            pl.BlockSpec(memory_space=pl.ANY),
            pl.BlockSpec(memory_space=pl.ANY),
            pl.BlockSpec((B, NB), lambda i, xr: (0, 0)),
        ],
        out_specs=[
            pl.BlockSpec((H, B, D), lambda i, xr: (0, 0, 0)),
        ],
        scratch_shapes=[
            pltpu.VMEM((P_DEV, B), jnp.float32),
            pltpu.VMEM((2, KB_TOK, GD_A), jnp.float32),
            pltpu.VMEM((2, KB_TOK, GD), jnp.float32),
            pltpu.VMEM((GB, 1), jnp.float32),
            pltpu.VMEM((GB, 1), jnp.float32),
            pltpu.VMEM((HG, B, D), jnp.float32),
            pltpu.VMEM((H, B, 1), jnp.float32),
            pltpu.VMEM((H, B, 1), jnp.float32),
            pltpu.VMEM((H, B, D), jnp.bfloat16),
            pltpu.VMEM((2 * H, B, 1), jnp.float32),
            pltpu.VMEM((3, H, B, D), jnp.bfloat16),
            pltpu.VMEM((3, 2 * H, B, 1), jnp.float32),
            pltpu.SemaphoreType.DMA((2, 2 * HG)),
            pltpu.SemaphoreType.DMA((G, 3)),
            pltpu.SemaphoreType.DMA((G, 3)),
            pltpu.SemaphoreType.DMA((3,)),
            pltpu.SemaphoreType.DMA((3,)),
        ],
    )
    return pl.pallas_call(
        _attn_body,
        grid_spec=grid_spec,
        out_shape=[
            jax.ShapeDtypeStruct((H, B, D), jnp.float32),
        ],
        compiler_params=pltpu.CompilerParams(
            dimension_semantics=("arbitrary",),
            collective_id=0),
    )(xarr, qbd, k, v, btv)


def kernel(Q, K, V, bt, lens):
    my_x = lax.axis_index("x")
    q = jnp.transpose(Q.reshape(B, H, D) * (SCALE * LOG2E),
                      (1, 0, 2))
    qg = q.reshape(G, HG, B, D)
    eye = jnp.eye(HG, dtype=q.dtype)
    qbd = (qg[:, :, :, None, :] * eye[None, :, None, :, None]).reshape(
        G, GB, GD)
    eye_b = jnp.tile(jnp.eye(B, dtype=q.dtype), (HG, 1))
    qbd = jnp.concatenate(
        [qbd, jnp.broadcast_to(eye_b[None], (G, GB, B))],
        axis=2).astype(jnp.bfloat16)
    k = K.reshape(P_LOCAL * BS, H, D)
    v = V.reshape(P_LOCAL * BS, H, D)
    valid = jnp.arange(NB, dtype=jnp.int32)[None, :] < lens[:, None]
    btv = jnp.where(valid, bt, -1)
    xarr = jnp.full((1,), my_x * N_KB, jnp.int32)
    (o,) = _attention(xarr, qbd, k, v, btv)
    return jnp.transpose(o, (1, 0, 2)).reshape(B, 1, H, D)


# device time: 37889 ns/iter; 1.2186x vs baseline; 1.2186x over previous
import jax
import jax.numpy as jnp
from jax import lax
from jax.experimental import pallas as pl
from jax.experimental.pallas import tpu as pltpu

B = 32
H = 16
D = 128
BS = 32
NB = 256
P_LOCAL = 256
P_DEV = 128
KB_PAGES = 64
KB_TOK = KB_PAGES * BS
N_KB = P_DEV // KB_PAGES
G = 4
HG = H // G
GB = HG * B
GD = HG * D
NEG = -1e30
LOG2E = 1.4426950408889634
SCALE = D ** -0.5
GD_A = GD + B
MESH = pl.DeviceIdType.MESH

TOTAL_STEPS = G * N_KB


def _attn_body(xref, qbd_ref, k_hbm, v_hbm, bt_ref, out_ref,
               bias_ref, expand_sc, kbuf, vbuf, m_sc, l_sc,
               acc_cur, s_acc, ml_send,
               r_acc, r_ml, dsems,
               asend, arecv, mlsend, mlrecv):
    step = pl.program_id(0)
    kb = step % N_KB
    my_x = lax.axis_index("x")
    my_y = lax.axis_index("y")
    peers = (
        (my_x, 1 - my_y),
        (1 - my_x, my_y),
        (1 - my_x, 1 - my_y),
    )

    def dma_descs(s, slot):
        g = s // N_KB
        row0 = (xref[0] + s % N_KB) * KB_TOK
        descs = []
        for t, (hbm, buf) in enumerate(((k_hbm, kbuf), (v_hbm, vbuf))):
            for a in range(HG):
                descs.append(pltpu.make_async_copy(
                    hbm.at[pl.ds(row0, KB_TOK), g * HG + a],
                    buf.at[slot, :, pl.ds(a * D, D)],
                    dsems.at[slot, t * HG + a]))
        return descs

    def acc_rdma(g, j):
        return pltpu.make_async_remote_copy(
            src_ref=s_acc.at[pl.ds(g * HG, HG)],
            dst_ref=r_acc.at[j, pl.ds(g * HG, HG)],
            send_sem=asend.at[g, j], recv_sem=arecv.at[g, j],
            device_id=peers[j], device_id_type=MESH)

    def ml_rdma(g, j):
        return pltpu.make_async_remote_copy(
            src_ref=ml_send.at[pl.ds(g * 2 * HG, 2 * HG)],
            dst_ref=r_ml.at[j, pl.ds(g * 2 * HG, 2 * HG)],
            send_sem=mlsend.at[g, j], recv_sem=mlrecv.at[g, j],
            device_id=peers[j], device_id_type=MESH)

    @pl.when(step == 0)
    def _prologue():
        for d in dma_descs(0, 0):
            d.start()
        barrier = pltpu.get_barrier_semaphore()
        for nbr in peers:
            pl.semaphore_signal(barrier, inc=1, device_id=nbr,
                                device_id_type=MESH)
        pl.semaphore_wait(barrier, 3)

    @pl.when(step + 1 < TOTAL_STEPS)
    def _prefetch():
        for d in dma_descs(step + 1, (step + 1) % 2):
            d.start()

    @pl.when(step == 0)
    def _bias():
        bt = bt_ref[...]
        base = my_y * P_LOCAL + xref[0] * KB_PAGES
        CH = 32
        for c in range(P_DEV // CH):
            pages = base + c * CH + lax.broadcasted_iota(
                jnp.int32, (CH, 1, 1), 0)
            eq = bt[None, :, :] == pages
            cnt = jnp.sum(jnp.where(eq, 1.0, 0.0), axis=2)
            bias_ref[c * CH:(c + 1) * CH, :] = jnp.where(
                cnt > 0.5, jnp.log2(cnt), NEG)

    @pl.when(kb == 0)
    def _init():
        m_sc[...] = jnp.full((GB, 1), NEG, jnp.float32)
        l_sc[...] = jnp.zeros((GB, 1), jnp.float32)

    @pl.when(step == 0)
    def _expand():
        rows = lax.broadcasted_iota(jnp.int32, (KB_PAGES, KB_TOK), 0)
        cols = lax.broadcasted_iota(jnp.int32, (KB_PAGES, KB_TOK), 1)
        expand_sc[...] = jnp.where(cols // BS == rows, 1.0, 0.0)

    bias_blk = bias_ref[pl.ds(kb * KB_PAGES, KB_PAGES), :]
    bias_cols = lax.dot_general(
        expand_sc[...], bias_blk, (((0,), (0,)), ((), ())),
        preferred_element_type=jnp.float32)

    slot = step % 2
    for d in dma_descs(step, slot):
        d.wait()
    kbuf[slot, :, GD:GD_A] = bias_cols

    g_dyn = step // N_KB
    kg = kbuf[slot].astype(jnp.bfloat16)
    vg = vbuf[slot].astype(jnp.bfloat16)
    s = lax.dot_general(
        qbd_ref[0], kg, (((1,), (1,)), ((), ())),
        preferred_element_type=jnp.float32)
    m_old = m_sc[...]
    m_new = jnp.maximum(m_old, jnp.max(s, axis=1, keepdims=True))
    p = jnp.exp2(s - m_new)
    corr = jnp.exp2(m_old - m_new)
    m_sc[...] = m_new
    l_sc[...] = l_sc[...] * corr + jnp.sum(p, axis=1, keepdims=True)
    o = lax.dot_general(
        p.astype(jnp.bfloat16), vg, (((1,), (0,)), ((), ())),
        preferred_element_type=jnp.float32)

    @pl.when(kb == 0)
    def _acc_first():
        for a in range(HG):
            acc_cur[a] = o[a * B:(a + 1) * B, a * D:(a + 1) * D]

    @pl.when(kb != 0)
    def _acc_rest():
        for a in range(HG):
            acc_cur[a] = (acc_cur[a] * corr[a * B:(a + 1) * B]
                          + o[a * B:(a + 1) * B, a * D:(a + 1) * D])

    @pl.when(kb == N_KB - 1)
    def _group_done():
        rows_g = pl.ds(g_dyn * HG, HG)
        ml_send[pl.ds(g_dyn * 2 * HG, HG)] = m_sc[...].reshape(HG, B, 1)
        ml_send[pl.ds(g_dyn * 2 * HG + HG, HG)] = l_sc[...].reshape(HG, B, 1)
        s_acc[rows_g] = acc_cur[...].astype(jnp.bfloat16)
        for j in range(3):
            acc_rdma(g_dyn, j).start()
            ml_rdma(g_dyn, j).start()

    def combine_group(g):
        for j in range(3):
            acc_rdma(g, j).wait()
            ml_rdma(g, j).wait()
        rows_g = pl.ds(g * HG, HG)
        m = ml_send[pl.ds(g * 2 * HG, HG)]
        r_m = [r_ml[j, pl.ds(g * 2 * HG, HG)] for j in range(3)]
        r_l = [r_ml[j, pl.ds(g * 2 * HG + HG, HG)] for j in range(3)]
        mt = jnp.maximum(jnp.maximum(m, r_m[0]),
                         jnp.maximum(r_m[1], r_m[2]))
        w = jnp.exp2(m - mt)
        lt = w * ml_send[pl.ds(g * 2 * HG + HG, HG)]
        ot = w * s_acc[rows_g].astype(jnp.float32)
        for j in range(3):
            w = jnp.exp2(r_m[j] - mt)
            lt = lt + w * r_l[j]
            ot = ot + w * r_acc[j, rows_g].astype(jnp.float32)
        out_ref[rows_g] = ot / lt

    @pl.when(jnp.logical_and(kb == N_KB - 1, g_dyn >= 1))
    def _combine_prev():
        combine_group(g_dyn - 1)

    @pl.when(step == TOTAL_STEPS - 1)
    def _finish():
        combine_group(G - 1)


def _attention(xarr, qbd, k, v, btv):
    grid_spec = pltpu.PrefetchScalarGridSpec(
        num_scalar_prefetch=1,
        grid=(TOTAL_STEPS,),
        in_specs=[
            pl.BlockSpec((1, GB, GD_A), lambda i, xr: (i // N_KB, 0, 0)),
            pl.BlockSpec(memory_space=pl.ANY),
            pl.BlockSpec(memory_space=pl.ANY),
            pl.BlockSpec((B, NB), lambda i, xr: (0, 0)),
        ],
        out_specs=[
            pl.BlockSpec((H, B, D), lambda i, xr: (0, 0, 0)),
        ],
        scratch_shapes=[
            pltpu.VMEM((P_DEV, B), jnp.float32),
            pltpu.VMEM((KB_PAGES, KB_TOK), jnp.float32),
            pltpu.VMEM((2, KB_TOK, GD_A), jnp.float32),
            pltpu.VMEM((2, KB_TOK, GD), jnp.float32),
            pltpu.VMEM((GB, 1), jnp.float32),
            pltpu.VMEM((GB, 1), jnp.float32),
            pltpu.VMEM((HG, B, D), jnp.float32),
            pltpu.VMEM((H, B, D), jnp.bfloat16),
            pltpu.VMEM((2 * H, B, 1), jnp.float32),
            pltpu.VMEM((3, H, B, D), jnp.bfloat16),
            pltpu.VMEM((3, 2 * H, B, 1), jnp.float32),
            pltpu.SemaphoreType.DMA((2, 2 * HG)),
            pltpu.SemaphoreType.DMA((G, 3)),
            pltpu.SemaphoreType.DMA((G, 3)),
            pltpu.SemaphoreType.DMA((G, 3)),
            pltpu.SemaphoreType.DMA((G, 3)),
        ],
    )
    return pl.pallas_call(
        _attn_body,
        grid_spec=grid_spec,
        out_shape=[
            jax.ShapeDtypeStruct((H, B, D), jnp.float32),
        ],
        compiler_params=pltpu.CompilerParams(
            dimension_semantics=("arbitrary",),
            collective_id=0),
    )(xarr, qbd, k, v, btv)


def kernel(Q, K, V, bt, lens):
    my_x = lax.axis_index("x")
    q = jnp.transpose(Q.reshape(B, H, D) * (SCALE * LOG2E),
                      (1, 0, 2))
    qg = q.reshape(G, HG, B, D)
    eye = jnp.eye(HG, dtype=q.dtype)
    qbd = (qg[:, :, :, None, :] * eye[None, :, None, :, None]).reshape(
        G, GB, GD)
    eye_b = jnp.tile(jnp.eye(B, dtype=q.dtype), (HG, 1))
    qbd = jnp.concatenate(
        [qbd, jnp.broadcast_to(eye_b[None], (G, GB, B))],
        axis=2).astype(jnp.bfloat16)
    k = K.reshape(P_LOCAL * BS, H, D)
    v = V.reshape(P_LOCAL * BS, H, D)
    valid = jnp.arange(NB, dtype=jnp.int32)[None, :] < lens[:, None]
    btv = jnp.where(valid, bt, -1)
    xarr = jnp.full((1,), my_x * N_KB, jnp.int32)
    (o,) = _attention(xarr, qbd, k, v, btv)
    return jnp.transpose(o, (1, 0, 2)).reshape(B, 1, H, D)
